# baseline (speedup 1.0000x reference)
import functools

import jax
import jax.numpy as jnp
from jax import lax
from jax.experimental import pallas as pl
from jax.experimental.pallas import tpu as pltpu
from jax.experimental.pallas import tpu_sc as plsc

DIM = 64
LANES = 16
QUARTERS = DIM // LANES
NUM_CORES = 2
NUM_SUBCORES = 16
NW = NUM_CORES * NUM_SUBCORES
GROUP = 16
SUBLANES = 8
MARGIN = 1.0

_GATHER_DNUMS = lax.GatherDimensionNumbers(
    offset_dims=(), collapsed_slice_dims=(0,), start_index_map=(0,))


def _lane_shuffle(x, perm):
    return lax.gather(
        x, perm[:, None], _GATHER_DNUMS, slice_sizes=(1,),
        mode=lax.GatherScatterMode.PROMISE_IN_BOUNDS)


def _make_transe(B):
    assert B % NW == 0
    per_w = B // NW
    assert per_w % GROUP == 0
    ngr = per_w // GROUP
    mesh = plsc.VectorSubcoreMesh(core_axis_name="c", subcore_axis_name="s")

    @functools.partial(
        pl.kernel,
        out_type=jax.ShapeDtypeStruct((NW, LANES), jnp.float32),
        mesh=mesh,
        scratch_types=[
            pltpu.VMEM((6, ngr, GROUP), jnp.int32),
            pltpu.VMEM((6, ngr, GROUP), jnp.int32),
            pltpu.VMEM((6, GROUP, DIM), jnp.float32),
            pltpu.VMEM((LANES,), jnp.float32),
            pltpu.SemaphoreType.DMA,
        ],
    )
    def transe_kernel(blk_hbm, sub_hbm, etab, rtab, out_hbm, blk_v, sub_v,
                      rows, ovec, sem):
        wid = lax.axis_index("s") * NUM_CORES + lax.axis_index("c")
        pltpu.sync_copy(blk_hbm.at[wid], blk_v)
        pltpu.sync_copy(sub_hbm.at[wid], sub_v)
        tabs = (etab, rtab, etab, etab, rtab, etab)

        def group_body(g, loss_vec):
            blk_vecs = [blk_v[j, g, :] for j in range(6)]
            sub_vecs = [sub_v[j, g, :] for j in range(6)]
            copies = [[
                pltpu.async_copy(
                    tabs[j].at[blk_vecs[j][k], sub_vecs[j][k]],
                    rows.at[j, k], sem)
                for j in range(6)] for k in range(GROUP)]
            for k in range(GROUP):
                for cp in copies[k]:
                    cp.wait()
                gap = None
                for q in range(QUARTERS):
                    sl = pl.ds(q * LANES, LANES)
                    p = jnp.abs(rows[0, k, sl] + rows[1, k, sl]
                                - rows[2, k, sl])
                    n = jnp.abs(rows[3, k, sl] + rows[4, k, sl]
                                - rows[5, k, sl])
                    gap = p - n if gap is None else gap + (p - n)
                s = gap
                for b in (8, 4, 2, 1):
                    perm = lax.iota(jnp.int32, LANES) ^ b
                    s = s + _lane_shuffle(s, perm)
                hinge = jnp.maximum(MARGIN + s, 0.0)
                lane_hit = lax.iota(jnp.int32, LANES) == k
                loss_vec = loss_vec + jnp.where(lane_hit, hinge, 0.0)
            return loss_vec

        loss_vec = lax.fori_loop(0, ngr, group_body,
                                 jnp.zeros((LANES,), jnp.float32))
        ovec[...] = loss_vec
        pltpu.sync_copy(ovec, out_hbm.at[wid])

    return transe_kernel


def kernel(positive_triples, negative_triples, entity_embeddings,
           relation_embeddings):
    B = positive_triples.shape[0]
    per_w = B // NW
    ngr = per_w // GROUP
    idx = jnp.stack(
        [
            positive_triples[:, 0],
            positive_triples[:, 1],
            positive_triples[:, 2],
            negative_triples[:, 0],
            negative_triples[:, 1],
            negative_triples[:, 2],
        ],
        axis=0,
    )
    idx = idx.reshape(6, NW, ngr, GROUP).transpose(1, 0, 2, 3)
    blk = idx >> 3
    sub = idx & 7
    etab3 = entity_embeddings.reshape(-1, SUBLANES, DIM)
    rtab3 = relation_embeddings.reshape(-1, SUBLANES, DIM)
    partials = _make_transe(B)(blk, sub, etab3, rtab3)
    return jnp.sum(partials) * (1.0 / B)


# speedup vs baseline: 1.0012x; 1.0012x over previous
import functools

import jax
import jax.numpy as jnp
from jax import lax
from jax.experimental import pallas as pl
from jax.experimental.pallas import tpu as pltpu
from jax.experimental.pallas import tpu_sc as plsc

DIM = 64
LANES = 16
QUARTERS = DIM // LANES
NUM_CORES = 2
NUM_SUBCORES = 16
NW = NUM_CORES * NUM_SUBCORES
GROUP = 64
SUBLANES = 8
MARGIN = 1.0

_GATHER_DNUMS = lax.GatherDimensionNumbers(
    offset_dims=(), collapsed_slice_dims=(0,), start_index_map=(0,))


def _lane_shuffle(x, perm):
    return lax.gather(
        x, perm[:, None], _GATHER_DNUMS, slice_sizes=(1,),
        mode=lax.GatherScatterMode.PROMISE_IN_BOUNDS)


def _make_transe(B):
    assert B % NW == 0
    per_w = B // NW
    assert per_w % GROUP == 0
    ngr = per_w // GROUP
    mesh = plsc.VectorSubcoreMesh(core_axis_name="c", subcore_axis_name="s")

    @functools.partial(
        pl.kernel,
        out_type=jax.ShapeDtypeStruct((NW, LANES), jnp.float32),
        mesh=mesh,
        scratch_types=[
            pltpu.VMEM((6, ngr, GROUP), jnp.int32),
            pltpu.VMEM((6, GROUP, DIM), jnp.float32),
            pltpu.VMEM((LANES,), jnp.float32),
            pltpu.SemaphoreType.DMA,
        ],
    )
    def transe_kernel(idx_hbm, etab, rtab, out_hbm, idx_v, rows, ovec, sem):
        wid = lax.axis_index("s") * NUM_CORES + lax.axis_index("c")
        pltpu.sync_copy(idx_hbm.at[wid], idx_v)
        tabs = (etab, rtab, etab, etab, rtab, etab)

        def group_body(g, loss_vec):
            copies = []
            for j in range(6):
                for k0 in range(0, GROUP, LANES):
                    idx_vec = idx_v[j, g, pl.ds(k0, LANES)]
                    for kk in range(LANES):
                        r = idx_vec[kk]
                        copies.append(pltpu.async_copy(
                            tabs[j].at[r >> 3, r & 7],
                            rows.at[j, k0 + kk], sem))
            for cp in copies:
                cp.wait()
            for k in range(GROUP):
                gap = None
                for q in range(QUARTERS):
                    sl = pl.ds(q * LANES, LANES)
                    p = jnp.abs(rows[0, k, sl] + rows[1, k, sl]
                                - rows[2, k, sl])
                    n = jnp.abs(rows[3, k, sl] + rows[4, k, sl]
                                - rows[5, k, sl])
                    gap = p - n if gap is None else gap + (p - n)
                s = gap
                for b in (8, 4, 2, 1):
                    perm = lax.iota(jnp.int32, LANES) ^ b
                    s = s + _lane_shuffle(s, perm)
                hinge = jnp.maximum(MARGIN + s, 0.0)
                lane_hit = lax.iota(jnp.int32, LANES) == (k % LANES)
                loss_vec = loss_vec + jnp.where(lane_hit, hinge, 0.0)
            return loss_vec

        loss_vec = lax.fori_loop(0, ngr, group_body,
                                 jnp.zeros((LANES,), jnp.float32))
        ovec[...] = loss_vec
        pltpu.sync_copy(ovec, out_hbm.at[wid])

    return transe_kernel


def kernel(positive_triples, negative_triples, entity_embeddings,
           relation_embeddings):
    B = positive_triples.shape[0]
    per_w = B // NW
    ngr = per_w // GROUP
    idx = jnp.stack(
        [
            positive_triples[:, 0],
            positive_triples[:, 1],
            positive_triples[:, 2],
            negative_triples[:, 0],
            negative_triples[:, 1],
            negative_triples[:, 2],
        ],
        axis=0,
    )
    idx = idx.reshape(6, NW, ngr, GROUP).transpose(1, 0, 2, 3)
    etab3 = entity_embeddings.reshape(-1, SUBLANES, DIM)
    rtab3 = relation_embeddings.reshape(-1, SUBLANES, DIM)
    partials = _make_transe(B)(idx, etab3, rtab3)
    return jnp.sum(partials) * (1.0 / B)


# speedup vs baseline: 1.0216x; 1.0204x over previous
import functools

import jax
import jax.numpy as jnp
from jax import lax
from jax.experimental import pallas as pl
from jax.experimental.pallas import tpu as pltpu
from jax.experimental.pallas import tpu_sc as plsc

DIM = 64
LANES = 16
QUARTERS = DIM // LANES
NUM_CORES = 2
NUM_SUBCORES = 16
NW = NUM_CORES * NUM_SUBCORES
GROUP = 16
SUBLANES = 8
MARGIN = 1.0

_GATHER_DNUMS = lax.GatherDimensionNumbers(
    offset_dims=(), collapsed_slice_dims=(0,), start_index_map=(0,))


def _lane_shuffle(x, perm):
    return lax.gather(
        x, perm[:, None], _GATHER_DNUMS, slice_sizes=(1,),
        mode=lax.GatherScatterMode.PROMISE_IN_BOUNDS)


def _make_transe(B):
    assert B % NW == 0
    per_w = B // NW
    assert per_w % GROUP == 0
    ngr = per_w // GROUP
    mesh = plsc.VectorSubcoreMesh(core_axis_name="c", subcore_axis_name="s")

    @functools.partial(
        pl.kernel,
        out_type=jax.ShapeDtypeStruct((NW, LANES), jnp.float32),
        mesh=mesh,
        scratch_types=[
            pltpu.VMEM((6, ngr, GROUP), jnp.int32),
            pltpu.VMEM((6, ngr, GROUP), jnp.int32),
            pltpu.VMEM((6, GROUP, DIM), jnp.float32),
            pltpu.VMEM((LANES,), jnp.float32),
            pltpu.SemaphoreType.DMA,
        ],
    )
    def transe_kernel(blk_hbm, sub_hbm, etab, rtab, out_hbm, blk_v, sub_v,
                      rows, ovec, sem):
        wid = lax.axis_index("s") * NUM_CORES + lax.axis_index("c")
        pltpu.sync_copy(blk_hbm.at[wid], blk_v)
        pltpu.sync_copy(sub_hbm.at[wid], sub_v)
        tabs = (etab, rtab, etab, etab, rtab, etab)

        def group_body(g, loss_vec):
            blk_vecs = [blk_v[j, g, :] for j in range(6)]
            sub_vecs = [sub_v[j, g, :] for j in range(6)]
            copies = []
            for j in range(6):
                for k in range(GROUP):
                    copies.append(pltpu.async_copy(
                        tabs[j].at[blk_vecs[j][k], sub_vecs[j][k]],
                        rows.at[j, k], sem))
            for cp in copies:
                cp.wait()
            for k in range(GROUP):
                gap = None
                for q in range(QUARTERS):
                    sl = pl.ds(q * LANES, LANES)
                    p = jnp.abs(rows[0, k, sl] + rows[1, k, sl]
                                - rows[2, k, sl])
                    n = jnp.abs(rows[3, k, sl] + rows[4, k, sl]
                                - rows[5, k, sl])
                    gap = p - n if gap is None else gap + (p - n)
                s = gap
                for b in (8, 4, 2, 1):
                    perm = lax.iota(jnp.int32, LANES) ^ b
                    s = s + _lane_shuffle(s, perm)
                hinge = jnp.maximum(MARGIN + s, 0.0)
                lane_hit = lax.iota(jnp.int32, LANES) == k
                loss_vec = loss_vec + jnp.where(lane_hit, hinge, 0.0)
            return loss_vec

        loss_vec = lax.fori_loop(0, ngr, group_body,
                                 jnp.zeros((LANES,), jnp.float32))
        ovec[...] = loss_vec
        pltpu.sync_copy(ovec, out_hbm.at[wid])

    return transe_kernel


def kernel(positive_triples, negative_triples, entity_embeddings,
           relation_embeddings):
    B = positive_triples.shape[0]
    per_w = B // NW
    ngr = per_w // GROUP
    idx = jnp.stack(
        [
            positive_triples[:, 0],
            positive_triples[:, 1],
            positive_triples[:, 2],
            negative_triples[:, 0],
            negative_triples[:, 1],
            negative_triples[:, 2],
        ],
        axis=0,
    )
    idx = idx.reshape(6, NW, ngr, GROUP).transpose(1, 0, 2, 3)
    blk = idx >> 3
    sub = idx & 7
    etab3 = entity_embeddings.reshape(-1, SUBLANES, DIM)
    rtab3 = relation_embeddings.reshape(-1, SUBLANES, DIM)
    partials = _make_transe(B)(blk, sub, etab3, rtab3)
    return jnp.sum(partials) * (1.0 / B)
